# Initial kernel scaffold; baseline (speedup 1.0000x reference)
import jax
import jax.numpy as jnp
from jax.experimental import pallas as pl


def kernel(text, lut_w, static_w, W1, b1, W2, b2):
    raise NotImplementedError("write your pallas kernel here")


# speedup vs baseline: 2.2917x; 2.2917x over previous
import functools

import jax
import jax.numpy as jnp
from jax import lax
from jax.experimental import pallas as pl
from jax.experimental.pallas import tpu as pltpu
from jax.experimental.pallas import tpu_sc as plsc

VOCAB = 100000
DIM = 128
B = 4096
L = 50
HID = 600
NCLS = 5

NC = 2
NS = 16
NW = NC * NS
B_PER_W = B // NW
ROWS_PER_GATHER = 104
N_CHUNK = B_PER_W // 2
IDX_PER_W = N_CHUNK * ROWS_PER_GATHER
NREG = DIM // 16


def _emb_pool_sc(text_flat, lut_w, static_w):
    mesh = plsc.VectorSubcoreMesh(core_axis_name="c", subcore_axis_name="s")

    @functools.partial(
        pl.kernel,
        out_type=jax.ShapeDtypeStruct((B, 2 * DIM), jnp.float32),
        mesh=mesh,
        scratch_types=[
            pltpu.VMEM((IDX_PER_W,), jnp.int32),
            pltpu.VMEM((ROWS_PER_GATHER, DIM), jnp.float32),
            pltpu.VMEM((B_PER_W, 2 * DIM), jnp.float32),
            pltpu.SemaphoreType.DMA,
        ],
    )
    def emb_kernel(text_hbm, lut_hbm, static_hbm, out_hbm, idx_v, rows_v, out_v, sem):
        wid = lax.axis_index("s") * NC + lax.axis_index("c")
        base_b = wid * B_PER_W
        pltpu.sync_copy(text_hbm.at[pl.ds(wid * IDX_PER_W, IDX_PER_W)], idx_v)

        def pool_table(table_hbm, col0):
            @pl.loop(0, N_CHUNK)
            def _(g):
                idx_slice = idx_v.at[pl.ds(g * ROWS_PER_GATHER, ROWS_PER_GATHER)]
                pltpu.async_copy(table_hbm.at[idx_slice], rows_v, sem).wait()
                for bi in range(2):
                    def body(l, carry):
                        return tuple(
                            carry[k] + rows_v[bi * L + l, pl.ds(k * 16, 16)]
                            for k in range(NREG)
                        )
                    acc = lax.fori_loop(
                        0, L, body,
                        tuple(jnp.zeros((16,), jnp.float32) for _ in range(NREG)),
                    )
                    for k in range(NREG):
                        out_v[2 * g + bi, pl.ds(col0 + k * 16, 16)] = acc[k]

        pool_table(lut_hbm, 0)
        pool_table(static_hbm, DIM)
        pltpu.sync_copy(out_v, out_hbm.at[pl.ds(base_b, B_PER_W)])

    return emb_kernel(text_flat, lut_w, static_w)


def _mlp_body(e_ref, w1_ref, b1_ref, w2_ref, b2_ref, out_ref):
    h = jnp.dot(e_ref[...], w1_ref[...], preferred_element_type=jnp.float32,
                precision=lax.Precision.HIGHEST)
    h = jnp.maximum(h + b1_ref[...], 0.0)
    logits = jnp.dot(h, w2_ref[...], preferred_element_type=jnp.float32,
                     precision=lax.Precision.HIGHEST)
    logits = logits + b2_ref[...]
    m = jnp.max(logits, axis=-1, keepdims=True)
    s = logits - m
    lse = jnp.log(jnp.sum(jnp.exp(s), axis=-1, keepdims=True))
    out_ref[...] = s - lse


def _mlp_tc(emb, W1, b1, W2, b2):
    BLK = 512
    grid = (B // BLK,)
    return pl.pallas_call(
        _mlp_body,
        grid=grid,
        in_specs=[
            pl.BlockSpec((BLK, 2 * DIM), lambda i: (i, 0)),
            pl.BlockSpec((2 * DIM, HID), lambda i: (0, 0)),
            pl.BlockSpec((1, HID), lambda i: (0, 0)),
            pl.BlockSpec((HID, NCLS), lambda i: (0, 0)),
            pl.BlockSpec((1, NCLS), lambda i: (0, 0)),
        ],
        out_specs=pl.BlockSpec((BLK, NCLS), lambda i: (i, 0)),
        out_shape=jax.ShapeDtypeStruct((B, NCLS), jnp.float32),
    )(emb, W1, b1, W2, b2)


def kernel(text, lut_w, static_w, W1, b1, W2, b2):
    text2 = text.reshape(B // 2, 2 * L)
    text2 = jnp.pad(text2, ((0, 0), (0, ROWS_PER_GATHER - 2 * L)))
    text_flat = text2.reshape(B // 2 * ROWS_PER_GATHER)
    emb = _emb_pool_sc(text_flat, lut_w, static_w)
    return _mlp_tc(emb, W1, b1.reshape(1, HID), W2, b2.reshape(1, NCLS))


# speedup vs baseline: 2.2918x; 1.0000x over previous
import functools

import jax
import jax.numpy as jnp
from jax import lax
from jax.experimental import pallas as pl
from jax.experimental.pallas import tpu as pltpu
from jax.experimental.pallas import tpu_sc as plsc

VOCAB = 100000
DIM = 128
B = 4096
L = 50
HID = 600
NCLS = 5

NC = 2
NS = 16
NW = NC * NS
B_PER_W = B // NW
ROWS_PER_GATHER = 104
N_CHUNK = B_PER_W // 2
IDX_PER_W = N_CHUNK * ROWS_PER_GATHER
NREG = DIM // 16


def _emb_pool_sc(text_flat, lut_w, static_w):
    mesh = plsc.VectorSubcoreMesh(core_axis_name="c", subcore_axis_name="s")

    @functools.partial(
        pl.kernel,
        out_type=jax.ShapeDtypeStruct((B, 2 * DIM), jnp.float32),
        mesh=mesh,
        scratch_types=[
            pltpu.VMEM((IDX_PER_W,), jnp.int32),
            pltpu.VMEM((ROWS_PER_GATHER, DIM), jnp.float32),
            pltpu.VMEM((ROWS_PER_GATHER, DIM), jnp.float32),
            pltpu.VMEM((B_PER_W, 2 * DIM), jnp.float32),
            pltpu.SemaphoreType.DMA,
            pltpu.SemaphoreType.DMA,
        ],
    )
    def emb_kernel(text_hbm, lut_hbm, static_hbm, out_hbm,
                   idx_v, rows0, rows1, out_v, sem0, sem1):
        wid = lax.axis_index("s") * NC + lax.axis_index("c")
        base_b = wid * B_PER_W
        pltpu.sync_copy(text_hbm.at[pl.ds(wid * IDX_PER_W, IDX_PER_W)], idx_v)

        def idx_slice(g):
            return idx_v.at[pl.ds(g * ROWS_PER_GATHER, ROWS_PER_GATHER)]

        def accumulate(rows_v, g, col0):
            for bi in range(2):
                def body(l, carry):
                    out = carry
                    for u in range(5):
                        out = tuple(
                            out[k] + rows_v[bi * L + l * 5 + u, pl.ds(k * 16, 16)]
                            for k in range(NREG)
                        )
                    return out

                acc = lax.fori_loop(
                    0, L // 5, body,
                    tuple(jnp.zeros((16,), jnp.float32) for _ in range(NREG)),
                )
                for k in range(NREG):
                    out_v[2 * g + bi, pl.ds(col0 + k * 16, 16)] = acc[k]

        def pool_table(table_hbm, col0):
            pltpu.async_copy(table_hbm.at[idx_slice(0)], rows0, sem0)
            pltpu.async_copy(table_hbm.at[idx_slice(1)], rows1, sem1)

            @pl.loop(0, N_CHUNK, step=2)
            def _(g):
                for b, rows_v, sem in ((0, rows0, sem0), (1, rows1, sem1)):
                    gg = g + b
                    pltpu.make_async_copy(
                        table_hbm.at[idx_slice(gg)], rows_v, sem).wait()
                    accumulate(rows_v, gg, col0)

                    @pl.when(gg + 2 < N_CHUNK)
                    def _():
                        pltpu.async_copy(
                            table_hbm.at[idx_slice(gg + 2)], rows_v, sem)

        pool_table(lut_hbm, 0)
        pool_table(static_hbm, DIM)
        pltpu.sync_copy(out_v, out_hbm.at[pl.ds(base_b, B_PER_W)])

    return emb_kernel(text_flat, lut_w, static_w)


def _mlp_body(e_ref, w1_ref, b1_ref, w2_ref, b2_ref, out_ref):
    h = jnp.dot(e_ref[...], w1_ref[...], preferred_element_type=jnp.float32,
                precision=lax.Precision.HIGHEST)
    h = jnp.maximum(h + b1_ref[...], 0.0)
    logits = jnp.dot(h, w2_ref[...], preferred_element_type=jnp.float32,
                     precision=lax.Precision.HIGHEST)
    logits = logits + b2_ref[...]
    m = jnp.max(logits, axis=-1, keepdims=True)
    s = logits - m
    lse = jnp.log(jnp.sum(jnp.exp(s), axis=-1, keepdims=True))
    out_ref[...] = s - lse


def _mlp_tc(emb, W1, b1, W2, b2):
    BLK = 512
    grid = (B // BLK,)
    return pl.pallas_call(
        _mlp_body,
        grid=grid,
        in_specs=[
            pl.BlockSpec((BLK, 2 * DIM), lambda i: (i, 0)),
            pl.BlockSpec((2 * DIM, HID), lambda i: (0, 0)),
            pl.BlockSpec((1, HID), lambda i: (0, 0)),
            pl.BlockSpec((HID, NCLS), lambda i: (0, 0)),
            pl.BlockSpec((1, NCLS), lambda i: (0, 0)),
        ],
        out_specs=pl.BlockSpec((BLK, NCLS), lambda i: (i, 0)),
        out_shape=jax.ShapeDtypeStruct((B, NCLS), jnp.float32),
    )(emb, W1, b1, W2, b2)


def kernel(text, lut_w, static_w, W1, b1, W2, b2):
    text2 = text.reshape(B // 2, 2 * L)
    text2 = jnp.pad(text2, ((0, 0), (0, ROWS_PER_GATHER - 2 * L)))
    text_flat = text2.reshape(B // 2 * ROWS_PER_GATHER)
    emb = _emb_pool_sc(text_flat, lut_w, static_w)
    return _mlp_tc(emb, W1, b1.reshape(1, HID), W2, b2.reshape(1, NCLS))


# speedup vs baseline: 2.2925x; 1.0003x over previous
import functools

import jax
import jax.numpy as jnp
from jax import lax
from jax.experimental import pallas as pl
from jax.experimental.pallas import tpu as pltpu
from jax.experimental.pallas import tpu_sc as plsc

VOCAB = 100000
DIM = 128
B = 4096
L = 50
HID = 600
NCLS = 5

NC = 2
NS = 16
NW = NC * NS
B_PER_W = B // NW
ROWS_PER_GATHER = 104
N_CHUNK = B_PER_W // 2
IDX_PER_W = N_CHUNK * ROWS_PER_GATHER
NREG = DIM // 16


def _emb_pool_sc(text_flat, lut_w, static_w):
    mesh = plsc.VectorSubcoreMesh(core_axis_name="c", subcore_axis_name="s")

    @functools.partial(
        pl.kernel,
        out_type=jax.ShapeDtypeStruct((B, 2 * DIM), jnp.float32),
        mesh=mesh,
        scratch_types=[
            pltpu.VMEM((IDX_PER_W,), jnp.int32),
            pltpu.VMEM((ROWS_PER_GATHER, DIM), jnp.float32),
            pltpu.VMEM((ROWS_PER_GATHER, DIM), jnp.float32),
            pltpu.VMEM((ROWS_PER_GATHER, DIM), jnp.float32),
            pltpu.VMEM((ROWS_PER_GATHER, DIM), jnp.float32),
            pltpu.VMEM((B_PER_W, 2 * DIM), jnp.float32),
            pltpu.SemaphoreType.DMA,
            pltpu.SemaphoreType.DMA,
            pltpu.SemaphoreType.DMA,
            pltpu.SemaphoreType.DMA,
        ],
    )
    def emb_kernel(text_hbm, lut_hbm, static_hbm, out_hbm,
                   idx_v, rows0, rows1, rows2, rows3, out_v,
                   sem0, sem1, sem2, sem3):
        wid = lax.axis_index("s") * NC + lax.axis_index("c")
        base_b = wid * B_PER_W
        pltpu.sync_copy(text_hbm.at[pl.ds(wid * IDX_PER_W, IDX_PER_W)], idx_v)

        def idx_slice(g):
            return idx_v.at[pl.ds(g * ROWS_PER_GATHER, ROWS_PER_GATHER)]

        def accumulate(rows_v, g, col0):
            for bi in range(2):
                def body(l, carry):
                    out = carry
                    for u in range(5):
                        out = tuple(
                            out[k] + rows_v[bi * L + l * 5 + u, pl.ds(k * 16, 16)]
                            for k in range(NREG)
                        )
                    return out

                acc = lax.fori_loop(
                    0, L // 5, body,
                    tuple(jnp.zeros((16,), jnp.float32) for _ in range(NREG)),
                )
                for k in range(NREG):
                    out_v[2 * g + bi, pl.ds(col0 + k * 16, 16)] = acc[k]

        ring = ((0, rows0, sem0), (1, rows1, sem1),
                (2, rows2, sem2), (3, rows3, sem3))
        NBUF = len(ring)

        def pool_table(table_hbm, col0):
            for b, rows_v, sem in ring:
                pltpu.async_copy(table_hbm.at[idx_slice(b)], rows_v, sem)

            @pl.loop(0, N_CHUNK, step=NBUF)
            def _(g):
                for b, rows_v, sem in ring:
                    gg = g + b
                    pltpu.make_async_copy(
                        table_hbm.at[idx_slice(gg)], rows_v, sem).wait()
                    accumulate(rows_v, gg, col0)

                    @pl.when(gg + NBUF < N_CHUNK)
                    def _():
                        pltpu.async_copy(
                            table_hbm.at[idx_slice(gg + NBUF)], rows_v, sem)

        pool_table(lut_hbm, 0)
        pool_table(static_hbm, DIM)
        pltpu.sync_copy(out_v, out_hbm.at[pl.ds(base_b, B_PER_W)])

    return emb_kernel(text_flat, lut_w, static_w)


def _mlp_body(e_ref, w1_ref, b1_ref, w2_ref, b2_ref, out_ref):
    h = jnp.dot(e_ref[...], w1_ref[...], preferred_element_type=jnp.float32,
                precision=lax.Precision.HIGHEST)
    h = jnp.maximum(h + b1_ref[...], 0.0)
    logits = jnp.dot(h, w2_ref[...], preferred_element_type=jnp.float32,
                     precision=lax.Precision.HIGHEST)
    logits = logits + b2_ref[...]
    m = jnp.max(logits, axis=-1, keepdims=True)
    s = logits - m
    lse = jnp.log(jnp.sum(jnp.exp(s), axis=-1, keepdims=True))
    out_ref[...] = s - lse


def _mlp_tc(emb, W1, b1, W2, b2):
    BLK = 512
    grid = (B // BLK,)
    return pl.pallas_call(
        _mlp_body,
        grid=grid,
        in_specs=[
            pl.BlockSpec((BLK, 2 * DIM), lambda i: (i, 0)),
            pl.BlockSpec((2 * DIM, HID), lambda i: (0, 0)),
            pl.BlockSpec((1, HID), lambda i: (0, 0)),
            pl.BlockSpec((HID, NCLS), lambda i: (0, 0)),
            pl.BlockSpec((1, NCLS), lambda i: (0, 0)),
        ],
        out_specs=pl.BlockSpec((BLK, NCLS), lambda i: (i, 0)),
        out_shape=jax.ShapeDtypeStruct((B, NCLS), jnp.float32),
    )(emb, W1, b1, W2, b2)


def kernel(text, lut_w, static_w, W1, b1, W2, b2):
    text2 = text.reshape(B // 2, 2 * L)
    text2 = jnp.pad(text2, ((0, 0), (0, ROWS_PER_GATHER - 2 * L)))
    text_flat = text2.reshape(B // 2 * ROWS_PER_GATHER)
    emb = _emb_pool_sc(text_flat, lut_w, static_w)
    return _mlp_tc(emb, W1, b1.reshape(1, HID), W2, b2.reshape(1, NCLS))
